# baseline (speedup 1.0000x reference)
import functools

import jax
import jax.numpy as jnp
from jax import lax
from jax.experimental import pallas as pl
from jax.experimental.pallas import tpu as pltpu
from jax.experimental.pallas import tpu_sc as plsc

_NUM_CLASSES = 100000
_D = 128
_B = 4096
_LAMBDA = 0.003

_NC = 2
_NS = 16
_L = 16
_NW = _NC * _NS
_BPW = _B // _NW
_COLS = _D // _L

_mesh = plsc.VectorSubcoreMesh(core_axis_name="c", subcore_axis_name="s")


@functools.partial(
    pl.kernel,
    out_type=(jax.ShapeDtypeStruct((_NW, _L), jnp.float32),
              jax.ShapeDtypeStruct((_NC, _L), jnp.float32)),
    mesh=_mesh,
    scratch_types=[
        pltpu.VMEM((_BPW,), jnp.int32),
        pltpu.VMEM((_BPW, _D), jnp.float32),
        pltpu.VMEM((_BPW, _D), jnp.float32),
        pltpu.VMEM((_L,), jnp.float32),
        pltpu.VMEM((_NS, _L), jnp.float32),
        pltpu.SemaphoreType.DMA,
        pltpu.SemaphoreType.DMA,
    ],
)
def _sc_center_loss(feat_hbm, lab_hbm, cent_hbm, stage_hbm, out_hbm,
                    idx_v, rows_v, feat_v, acc_v, red_v, sem_g, sem_f):
    cid = lax.axis_index("c")
    sid = lax.axis_index("s")
    wid = cid * _NS + sid
    base = wid * _BPW

    pltpu.sync_copy(lab_hbm.at[pl.ds(base, _BPW)], idx_v)
    cp_f = pltpu.async_copy(feat_hbm.at[pl.ds(base, _BPW)], feat_v, sem_f)
    cp_g = pltpu.async_copy(cent_hbm.at[idx_v], rows_v, sem_g)
    cp_f.wait()
    cp_g.wait()

    def row_body(i, acc):
        for j in range(_COLS):
            f = feat_v[i, pl.ds(j * _L, _L)]
            c = rows_v[i, pl.ds(j * _L, _L)]
            d = f - c
            acc = acc + d * d
        return acc

    acc = lax.fori_loop(0, _BPW, row_body, jnp.zeros((_L,), jnp.float32))
    acc_v[...] = acc

    pltpu.sync_copy(acc_v, stage_hbm.at[wid])
    plsc.subcore_barrier()

    @pl.when(sid == 0)
    def _():
        pltpu.sync_copy(stage_hbm.at[pl.ds(cid * _NS, _NS)], red_v)
        total = red_v[0, :]
        for t in range(1, _NS):
            total = total + red_v[t, :]
        lanes = lax.iota(jnp.int32, _L)
        dnums = lax.GatherDimensionNumbers(
            offset_dims=(), collapsed_slice_dims=(0,), start_index_map=(0,))
        for s in (8, 4, 2, 1):
            perm = lax.rem(lanes + s, _L)
            total = total + lax.gather(
                total, perm[:, None], dnums, slice_sizes=(1,),
                mode=lax.GatherScatterMode.PROMISE_IN_BOUNDS)
        acc_v[...] = total * (_LAMBDA / _B)
        pltpu.sync_copy(acc_v, out_hbm.at[cid])


@jax.jit
def kernel(features, labels, centers):
    _, per_core = _sc_center_loss(features, labels.astype(jnp.int32), centers)
    return per_core[0, 0] + per_core[1, 0]


# speedup vs baseline: 1.0729x; 1.0729x over previous
import functools

import jax
import jax.numpy as jnp
from jax import lax
from jax.experimental import pallas as pl
from jax.experimental.pallas import tpu as pltpu
from jax.experimental.pallas import tpu_sc as plsc

_NUM_CLASSES = 100000
_D = 128
_B = 4096
_LAMBDA = 0.003

_NC = 2
_NS = 16
_L = 16
_NW = _NC * _NS
_BPW = _B // _NW
_COLS = _D // _L
_C = 32
_NCH = _BPW // _C

_mesh = plsc.VectorSubcoreMesh(core_axis_name="c", subcore_axis_name="s")


@functools.partial(
    pl.kernel,
    out_type=jax.ShapeDtypeStruct((_NW, _L), jnp.float32),
    mesh=_mesh,
    scratch_types=[
        pltpu.VMEM((_BPW,), jnp.int32),
        pltpu.VMEM((_C, _D), jnp.float32),
        pltpu.VMEM((_C, _D), jnp.float32),
        pltpu.VMEM((_C, _D), jnp.float32),
        pltpu.VMEM((_C, _D), jnp.float32),
        pltpu.VMEM((_L,), jnp.float32),
        pltpu.SemaphoreType.DMA,
        pltpu.SemaphoreType.DMA,
        pltpu.SemaphoreType.DMA,
        pltpu.SemaphoreType.DMA,
    ],
)
def _sc_partial_sums(feat_hbm, lab_hbm, cent_hbm, out_hbm,
                     idx_v, rows0, rows1, feat0, feat1, acc_v,
                     sg0, sg1, sf0, sf1):
    wid = lax.axis_index("c") * _NS + lax.axis_index("s")
    base = wid * _BPW

    rows_bufs = (rows0, rows1)
    feat_bufs = (feat0, feat1)
    g_sems = (sg0, sg1)
    f_sems = (sf0, sf1)

    pltpu.sync_copy(lab_hbm.at[pl.ds(base, _BPW)], idx_v)

    copies = {}

    def issue(k):
        s = k % 2
        copies[k] = (
            pltpu.async_copy(feat_hbm.at[pl.ds(base + k * _C, _C)],
                             feat_bufs[s], f_sems[s]),
            pltpu.async_copy(cent_hbm.at[idx_v.at[pl.ds(k * _C, _C)]],
                             rows_bufs[s], g_sems[s]),
        )

    issue(0)
    acc = jnp.zeros((_L,), jnp.float32)
    for k in range(_NCH):
        if k + 1 < _NCH:
            issue(k + 1)
        cp_f, cp_g = copies.pop(k)
        cp_f.wait()
        cp_g.wait()
        s = k % 2
        fbuf = feat_bufs[s]
        rbuf = rows_bufs[s]

        def chunk_body(i, acc, fbuf=fbuf, rbuf=rbuf):
            for j in range(_COLS):
                f = fbuf[i, pl.ds(j * _L, _L)]
                c = rbuf[i, pl.ds(j * _L, _L)]
                d = f - c
                acc = acc + d * d
            return acc

        acc = lax.fori_loop(0, _C, chunk_body, acc)

    acc_v[...] = acc
    pltpu.sync_copy(acc_v, out_hbm.at[wid])


def _tc_finish(p_ref, o_ref):
    o_ref[0, 0] = jnp.sum(p_ref[...]) * (_LAMBDA / _B)


_finish_call = pl.pallas_call(
    _tc_finish,
    out_shape=jax.ShapeDtypeStruct((1, 1), jnp.float32),
    out_specs=pl.BlockSpec(memory_space=pltpu.SMEM),
)


@jax.jit
def kernel(features, labels, centers):
    partials = _sc_partial_sums(features, labels.astype(jnp.int32), centers)
    return _finish_call(partials)[0, 0]


# speedup vs baseline: 1.1133x; 1.0377x over previous
import functools

import jax
import jax.numpy as jnp
from jax import lax
from jax.experimental import pallas as pl
from jax.experimental.pallas import tpu as pltpu
from jax.experimental.pallas import tpu_sc as plsc

_NUM_CLASSES = 100000
_D = 128
_B = 4096
_LAMBDA = 0.003

_NC = 2
_NS = 16
_L = 16
_NW = _NC * _NS
_BPW = _B // _NW
_COLS = _D // _L
_C = 64
_NCH = _BPW // _C

_mesh = plsc.VectorSubcoreMesh(core_axis_name="c", subcore_axis_name="s")


@functools.partial(
    pl.kernel,
    out_type=jax.ShapeDtypeStruct((_NW, _L), jnp.float32),
    mesh=_mesh,
    scratch_types=[
        pltpu.VMEM((_BPW,), jnp.int32),
        pltpu.VMEM((_C, _D), jnp.float32),
        pltpu.VMEM((_C, _D), jnp.float32),
        pltpu.VMEM((_BPW, _D), jnp.float32),
        pltpu.VMEM((_L,), jnp.float32),
        pltpu.SemaphoreType.DMA,
        pltpu.SemaphoreType.DMA,
        pltpu.SemaphoreType.DMA,
    ],
)
def _sc_partial_sums(feat_hbm, lab_hbm, cent_hbm, out_hbm,
                     idx_v, rows0, rows1, feat_v, acc_v,
                     sg0, sg1, sf):
    wid = lax.axis_index("c") * _NS + lax.axis_index("s")
    base = wid * _BPW

    rows_bufs = (rows0, rows1)
    g_sems = (sg0, sg1)

    cp_f = pltpu.async_copy(feat_hbm.at[pl.ds(base, _BPW)], feat_v, sf)
    pltpu.sync_copy(lab_hbm.at[pl.ds(base, _BPW)], idx_v)

    copies = {}

    def issue(k):
        s = k % 2
        copies[k] = pltpu.async_copy(
            cent_hbm.at[idx_v.at[pl.ds(k * _C, _C)]], rows_bufs[s], g_sems[s])

    issue(0)
    if _NCH > 1:
        issue(1)
    cp_f.wait()
    acc = jnp.zeros((_L,), jnp.float32)
    for k in range(_NCH):
        copies.pop(k).wait()
        if k + 2 < _NCH:
            issue(k + 2)
        rbuf = rows_bufs[k % 2]
        off = k * _C

        def chunk_body(i, acc, rbuf=rbuf, off=off):
            for j in range(_COLS):
                f = feat_v[off + i, pl.ds(j * _L, _L)]
                c = rbuf[i, pl.ds(j * _L, _L)]
                d = f - c
                acc = acc + d * d
            return acc

        acc = lax.fori_loop(0, _C, chunk_body, acc)

    acc_v[...] = acc
    pltpu.sync_copy(acc_v, out_hbm.at[wid])


def _tc_finish(p_ref, o_ref):
    o_ref[0, 0] = jnp.sum(p_ref[...]) * (_LAMBDA / _B)


_finish_call = pl.pallas_call(
    _tc_finish,
    out_shape=jax.ShapeDtypeStruct((1, 1), jnp.float32),
    out_specs=pl.BlockSpec(memory_space=pltpu.SMEM),
)


@jax.jit
def kernel(features, labels, centers):
    partials = _sc_partial_sums(features, labels.astype(jnp.int32), centers)
    return _finish_call(partials)[0, 0]
